# baseline (speedup 1.0000x reference)
import jax
import jax.numpy as jnp
from jax.experimental import pallas as pl

N = 1024
L = 27.4
K = 12
CUTOFF = 15.0
MAXNBR = 32


def kernel(positions, lattice):
    n = positions.shape[0]
    off = jnp.array([[i, j, l] for i in (-1, 0, 1) for j in (-1, 0, 1) for l in (-1, 0, 1)],
                    dtype=positions.dtype)
    shifts = off @ lattice
    images = (positions[None, :, :] + shifts[:, None, :]).reshape(-1, 3)
    p2 = jnp.sum(positions * positions, axis=1)
    q2 = jnp.sum(images * images, axis=1)
    def _dot_blk(p_ref, im_ref, o_ref):
        o_ref[...] = jax.lax.dot_general(
            p_ref[...], im_ref[...], (((1,), (1,)), ((), ())),
            preferred_element_type=jnp.float32)

    BC = 3456
    dot_pl = pl.pallas_call(
        _dot_blk,
        grid=(images.shape[0] // BC,),
        in_specs=[pl.BlockSpec((n, 3), lambda c: (0, 0)),
                  pl.BlockSpec((BC, 3), lambda c: (c, 0))],
        out_specs=pl.BlockSpec((n, BC), lambda c: (0, c)),
        out_shape=jax.ShapeDtypeStruct((n, images.shape[0]), jnp.float32),
    )(positions, images)
    d2 = p2[:, None] + q2[None, :] - 2.0 * dot_pl
    d2 = jnp.where(d2 < 1e-8, jnp.inf, d2)
    neg_d2, idx = jax.lax.top_k(-d2, MAXNBR)
    d = jnp.sqrt(jnp.maximum(-neg_d2, 0.0))
    kth = d[:, K - 1]
    shell_r = jnp.minimum(kth * (1.0 + 1e-4), jnp.asarray(CUTOFF, d.dtype))
    edge_mask = (d <= shell_r[:, None]) & jnp.isfinite(d)
    dst = (idx % n).astype(jnp.int32)
    edge_vec = images[idx] - positions[:, None, :]
    edge_vec = edge_vec * edge_mask[..., None].astype(edge_vec.dtype)
    return dst, edge_vec, edge_mask


# speedup vs baseline: 1.0501x; 1.0501x over previous
import functools

import jax
import jax.numpy as jnp
from jax import lax
from jax.experimental import pallas as pl
from jax.experimental.pallas import tpu as pltpu
from jax.experimental.pallas import tpu_sc as plsc

_N = 1024
_NIMG = 27 * _N
_K = 12
_CUTOFF = 15.0
_MAXNBR = 32
_BC_A = 3456


def _a_body(p_ref, im_ref, p2_ref, q2_ref, o_ref):
    dot = lax.dot_general(p_ref[...], im_ref[...], (((1,), (1,)), ((), ())),
                          preferred_element_type=jnp.float32)
    d2 = (p2_ref[...] + q2_ref[...]) - 2.0 * dot
    o_ref[...] = jnp.where(d2 < 1e-8, jnp.inf, d2)


def _stage_a(positions, images, p2, q2):
    return pl.pallas_call(
        _a_body,
        grid=(_NIMG // _BC_A,),
        in_specs=[
            pl.BlockSpec((_N, 3), lambda c: (0, 0)),
            pl.BlockSpec((_BC_A, 3), lambda c: (c, 0)),
            pl.BlockSpec((_N, 1), lambda c: (0, 0)),
            pl.BlockSpec((1, _BC_A), lambda c: (0, c)),
        ],
        out_specs=pl.BlockSpec((_N, _BC_A), lambda c: (0, c)),
        out_shape=jax.ShapeDtypeStruct((_N, _NIMG), jnp.float32),
    )(positions, images, p2[:, None], q2[None, :])


_ROWS_PER_W = 32
_NCHUNK = _NIMG // 16


def _sc_topk(d2):
    mesh = plsc.VectorSubcoreMesh(core_axis_name="c", subcore_axis_name="s")

    @functools.partial(
        pl.kernel,
        mesh=mesh,
        out_type=[
            jax.ShapeDtypeStruct((_N, _MAXNBR), jnp.float32),
            jax.ShapeDtypeStruct((_N, _MAXNBR), jnp.int32),
        ],
        scratch_types=[
            pltpu.VMEM((1, _NIMG), jnp.float32),
            pltpu.VMEM((1, _MAXNBR), jnp.float32),
            pltpu.VMEM((1, _MAXNBR), jnp.int32),
            pltpu.VMEM((1, _MAXNBR), jnp.float32),
            pltpu.VMEM((1, _MAXNBR), jnp.int32),
            pltpu.VMEM((1, 16), jnp.float32),
            pltpu.VMEM((1, 16), jnp.int32),
        ],
    )
    def topk_kernel(d2_hbm, outd_hbm, outi_hbm, rowbuf, rd_v, ri_v,
                    bufd, bufi, thrv, procv):
        wid = lax.axis_index("s") * 2 + lax.axis_index("c")
        iota = lax.broadcasted_iota(jnp.int32, (16,), 0)
        inf16 = jnp.full((16,), jnp.inf, jnp.float32)

        def _bfmax(x):
            for sh in (8, 4, 2, 1):
                x = jnp.maximum(x, x.at[iota ^ sh].get(mode="promise_in_bounds"))
            return x

        def _bfmin(x):
            for sh in (8, 4, 2, 1):
                x = jnp.minimum(x, x.at[iota ^ sh].get(mode="promise_in_bounds"))
            return x

        def _insert(v, cbase, guarded):
            td0_ = bufd[0, pl.ds(0, 16)]
            td1_ = bufd[0, pl.ds(16, 16)]
            ti0_ = bufi[0, pl.ds(0, 16)]
            ti1_ = bufi[0, pl.ds(16, 16)]
            thr_ = thrv[0, pl.ds(0, 16)]
            cmpv = (v < thr_) & (procv[0, pl.ds(0, 16)] == 0)
            lane = _bfmin(jnp.where(cmpv, iota, 16))
            qualv = lane < 16
            lane = jnp.minimum(lane, 15)
            procv[0, pl.ds(0, 16)] = jnp.where(
                iota == lane, 1, procv[0, pl.ds(0, 16)])
            dv = v.at[lane].get(mode="promise_in_bounds")
            di = lane + cbase
            maxv = _bfmax(jnp.maximum(td0_, td1_))
            eq0 = td0_ == maxv
            eq1 = td1_ == maxv
            mi = _bfmax(jnp.maximum(jnp.where(eq0, ti0_, -1),
                                    jnp.where(eq1, ti1_, -1)))
            rep0 = eq0 & (ti0_ == mi)
            rep1 = eq1 & (ti1_ == mi)
            if guarded:
                rep0 = rep0 & qualv
                rep1 = rep1 & qualv
            td0n = jnp.where(rep0, dv, td0_)
            td1n = jnp.where(rep1, dv, td1_)
            thrn = _bfmax(jnp.maximum(td0n, td1n))
            bufd[0, pl.ds(0, 16)] = td0n
            bufd[0, pl.ds(16, 16)] = td1n
            bufi[0, pl.ds(0, 16)] = jnp.where(rep0, di, ti0_)
            bufi[0, pl.ds(16, 16)] = jnp.where(rep1, di, ti1_)
            thrv[0, pl.ds(0, 16)] = thrn

        def _chunk(cbase):
            v = rowbuf[0, pl.ds(cbase, 16)]
            vmin = _bfmin(v)
            hit = vmin[0] < thrv[0, 0:16][0]

            @pl.when(hit)
            def fast():
                v_ = rowbuf[0, pl.ds(cbase, 16)]
                procv[0, pl.ds(0, 16)] = jnp.zeros((16,), jnp.int32)
                thr1 = thrv[0, pl.ds(0, 16)]
                lane0 = _bfmin(jnp.where(v_ < thr1, iota, 16))
                _insert(v_, cbase, False)
                vrem = jnp.where(iota == lane0, jnp.inf, v_)
                hit2 = _bfmin(vrem)[0] < thrv[0, 0:16][0]

                @pl.when(hit2)
                def slow():
                    def sbody(k, c):
                        _insert(rowbuf[0, pl.ds(cbase, 16)], cbase, True)
                        return c

                    lax.fori_loop(0, 15, sbody, 0)

        def row_body(r, carry):
            row = wid * _ROWS_PER_W + r
            pltpu.sync_copy(d2_hbm.at[pl.ds(row, 1)], rowbuf)
            bufd[0, pl.ds(0, 16)] = inf16
            bufd[0, pl.ds(16, 16)] = inf16
            bufi[0, pl.ds(0, 16)] = iota + (1 << 20)
            bufi[0, pl.ds(16, 16)] = iota + (1 << 20) + 16
            thrv[0, pl.ds(0, 16)] = inf16

            def group_body(g, c):
                base = g * 64
                v0 = rowbuf[0, pl.ds(base, 16)]
                v1 = rowbuf[0, pl.ds(base + 16, 16)]
                v2 = rowbuf[0, pl.ds(base + 32, 16)]
                v3 = rowbuf[0, pl.ds(base + 48, 16)]
                mn = jnp.minimum(jnp.minimum(v0, v1), jnp.minimum(v2, v3))
                gmin = _bfmin(mn)

                @pl.when(gmin[0] < thrv[0, 0:16][0])
                def gslow():
                    _chunk(base)
                    _chunk(base + 16)
                    _chunk(base + 32)
                    _chunk(base + 48)

                return c

            lax.fori_loop(0, _NIMG // 64, group_body, 0)

            def sel_body(k, c):
                td0_ = bufd[0, pl.ds(0, 16)]
                td1_ = bufd[0, pl.ds(16, 16)]
                ti0_ = bufi[0, pl.ds(0, 16)]
                ti1_ = bufi[0, pl.ds(16, 16)]
                mv = _bfmin(jnp.minimum(td0_, td1_))
                eq0 = td0_ == mv
                eq1 = td1_ == mv
                mi = _bfmin(jnp.minimum(jnp.where(eq0, ti0_, 1 << 24),
                                        jnp.where(eq1, ti1_, 1 << 24)))
                w0 = iota == k
                w1 = iota == (k - 16)
                rd_v[0, pl.ds(0, 16)] = jnp.where(w0, mv, rd_v[0, pl.ds(0, 16)])
                rd_v[0, pl.ds(16, 16)] = jnp.where(w1, mv, rd_v[0, pl.ds(16, 16)])
                ri_v[0, pl.ds(0, 16)] = jnp.where(w0, mi, ri_v[0, pl.ds(0, 16)])
                ri_v[0, pl.ds(16, 16)] = jnp.where(w1, mi, ri_v[0, pl.ds(16, 16)])
                rm0 = eq0 & (ti0_ == mi)
                rm1 = eq1 & (ti1_ == mi)
                bufd[0, pl.ds(0, 16)] = jnp.where(rm0, jnp.inf, td0_)
                bufd[0, pl.ds(16, 16)] = jnp.where(rm1, jnp.inf, td1_)
                bufi[0, pl.ds(0, 16)] = jnp.where(rm0, 1 << 22, ti0_)
                bufi[0, pl.ds(16, 16)] = jnp.where(rm1, 1 << 22, ti1_)
                return c

            lax.fori_loop(0, _MAXNBR, sel_body, 0)

            pltpu.sync_copy(rd_v, outd_hbm.at[pl.ds(row, 1)])
            pltpu.sync_copy(ri_v, outi_hbm.at[pl.ds(row, 1)])
            return carry

        lax.fori_loop(0, _ROWS_PER_W, row_body, 0)

    return topk_kernel(d2)


_BR_C = 64


def _c_body(d2_ref, idx_ref, d2f_ref, kf_ref, jf_ref, sf_ref, pos_ref,
            sh_ref, prep_ref, dst_ref, mask_ref, ev_ref):
    d2b = d2_ref[...]
    d = jnp.sqrt(jnp.maximum(d2b, 0.0))
    shell = jnp.minimum(d[:, _K - 1:_K] * (1.0 + 1e-4), jnp.float32(_CUTOFF))
    mask = (d <= shell) & jnp.isfinite(d)
    mask_ref[...] = mask.astype(jnp.float32)
    idxb = idx_ref[...]
    sid = idxb // _N
    dst_ref[...] = idxb - sid * _N

    f = _BR_C * _MAXNBR
    oh_j = (lax.broadcasted_iota(jnp.int32, (f, _N), 1)
            == jf_ref[...]).astype(jnp.float32)
    g = lax.dot_general(oh_j, pos_ref[...], (((1,), (0,)), ((), ())),
                        preferred_element_type=jnp.float32,
                        precision=lax.Precision.HIGHEST)
    oh_s = (lax.broadcasted_iota(jnp.int32, (f, 27), 1)
            == sf_ref[...]).astype(jnp.float32)
    sh = lax.dot_general(oh_s, sh_ref[...], (((1,), (0,)), ((), ())),
                         preferred_element_type=jnp.float32,
                         precision=lax.Precision.HIGHEST)
    ev = (g + sh) - prep_ref[...]
    df = jnp.sqrt(jnp.maximum(d2f_ref[...], 0.0))
    shf = jnp.minimum(jnp.sqrt(jnp.maximum(kf_ref[...], 0.0)) * (1.0 + 1e-4),
                      jnp.float32(_CUTOFF))
    mf = (df <= shf) & jnp.isfinite(df)
    ev_ref[...] = ev * mf.astype(jnp.float32)


def _stage_c(d2sel, idxsel, positions, shifts):
    f_all = _N * _MAXNBR
    d2_flat = d2sel.reshape(f_all, 1)
    kth_flat = jnp.repeat(d2sel[:, _K - 1], _MAXNBR).reshape(f_all, 1)
    sid_flat = (idxsel // _N).reshape(f_all, 1)
    j_flat = (idxsel - (idxsel // _N) * _N).reshape(f_all, 1)
    pos_rep = jnp.repeat(positions, _MAXNBR, axis=0)
    fb = _BR_C * _MAXNBR
    grid = _N // _BR_C
    dst, maskf, ev = pl.pallas_call(
        _c_body,
        grid=(grid,),
        in_specs=[
            pl.BlockSpec((_BR_C, _MAXNBR), lambda r: (r, 0)),
            pl.BlockSpec((_BR_C, _MAXNBR), lambda r: (r, 0)),
            pl.BlockSpec((fb, 1), lambda r: (r, 0)),
            pl.BlockSpec((fb, 1), lambda r: (r, 0)),
            pl.BlockSpec((fb, 1), lambda r: (r, 0)),
            pl.BlockSpec((fb, 1), lambda r: (r, 0)),
            pl.BlockSpec((_N, 3), lambda r: (0, 0)),
            pl.BlockSpec((27, 3), lambda r: (0, 0)),
            pl.BlockSpec((fb, 3), lambda r: (r, 0)),
        ],
        out_specs=[
            pl.BlockSpec((_BR_C, _MAXNBR), lambda r: (r, 0)),
            pl.BlockSpec((_BR_C, _MAXNBR), lambda r: (r, 0)),
            pl.BlockSpec((fb, 3), lambda r: (r, 0)),
        ],
        out_shape=[
            jax.ShapeDtypeStruct((_N, _MAXNBR), jnp.int32),
            jax.ShapeDtypeStruct((_N, _MAXNBR), jnp.float32),
            jax.ShapeDtypeStruct((f_all, 3), jnp.float32),
        ],
    )(d2sel, idxsel, d2_flat, kth_flat, j_flat, sid_flat, positions, shifts,
      pos_rep)
    return dst, maskf, ev


def kernel(positions, lattice):
    off = jnp.array([[i, j, l] for i in (-1, 0, 1) for j in (-1, 0, 1)
                     for l in (-1, 0, 1)], dtype=positions.dtype)
    shifts = off @ lattice
    images = (positions[None, :, :] + shifts[:, None, :]).reshape(-1, 3)
    p2 = jnp.sum(positions * positions, axis=1)
    q2 = jnp.sum(images * images, axis=1)
    d2 = _stage_a(positions, images, p2, q2)
    d2sel, idxsel = _sc_topk(d2)
    dst, maskf, ev = _stage_c(d2sel, idxsel, positions, shifts)
    edge_vec = ev.reshape(_N, _MAXNBR, 3)
    edge_mask = maskf.astype(bool)
    return dst, edge_vec, edge_mask
